# Initial kernel scaffold; baseline (speedup 1.0000x reference)
import jax
import jax.numpy as jnp
from jax.experimental import pallas as pl


def kernel(g_i, mention_scores, mention_ids, antecedent_ids, distance_ids, genre_ids, speaker_ids, W_dist, W_genre, W_speaker, W1, b1, W2, b2, W3, b3):
    raise NotImplementedError("write your pallas kernel here")


# speedup vs baseline: 4.8646x; 4.8646x over previous
import jax
import jax.numpy as jnp
from jax import lax
from jax.experimental import pallas as pl
from jax.experimental.pallas import tpu as pltpu
from jax.experimental.pallas import tpu_sc as plsc

N = 2048
P = 16384
D = 1024
HID = 150
HP = 256
B = 256
NBLK = P // B
NC, NS = 2, 16
NW = NC * NS
PPW = P // NW
C = 16
NCH = PPW // C
F32 = jnp.float32


def _prep_kernel(g_ref, w1a_ref, w1b_ref, ms_ref, b1_ref, e48_ref, w1d_ref,
                 gm_ref, ga_ref, phi_ref):
    g = g_ref[...]
    col = lax.broadcasted_iota(jnp.int32, (1, HP), 1)
    is150 = (col == HID).astype(F32)
    ms = ms_ref[...]
    gm_ref[...] = (jnp.dot(g, w1a_ref[...], preferred_element_type=F32)
                   + b1_ref[...] + ms * is150)
    ga_ref[...] = (jnp.dot(g, w1b_ref[...], preferred_element_type=F32)
                   + ms * is150)
    phi_ref[...] = jnp.dot(e48_ref[...], w1d_ref[...], preferred_element_type=F32)


def _sc_body(g_hbm, gm_hbm, ga_hbm, mid_hbm, aid_hbm, x_hbm, aff_hbm,
             midx, aidx, gi, gj, gm, ga, s_gi, s_gj, s_gm, s_ga, s_wx, s_wa):
    wid = lax.axis_index("s") * NC + lax.axis_index("c")
    base = wid * PPW
    pltpu.sync_copy(mid_hbm.at[pl.ds(base, PPW)], midx)
    pltpu.sync_copy(aid_hbm.at[pl.ds(base, PPW)], aidx)

    def gather_descs(k):
        off = (k % 2) * C
        i_idx = midx.at[pl.ds(k * C, C)]
        j_idx = aidx.at[pl.ds(k * C, C)]
        return (
            (g_hbm.at[i_idx], gi.at[pl.ds(off, C)], s_gi),
            (g_hbm.at[j_idx], gj.at[pl.ds(off, C)], s_gj),
            (gm_hbm.at[i_idx], gm.at[pl.ds(off, C)], s_gm),
            (ga_hbm.at[j_idx], ga.at[pl.ds(off, C)], s_ga),
        )

    def write_descs(k):
        off = (k % 2) * C
        row = base + k * C
        return (
            (gi.at[pl.ds(off, C)], x_hbm.at[pl.ds(row, C)], s_wx),
            (gm.at[pl.ds(off, C)], aff_hbm.at[pl.ds(row, C)], s_wa),
        )

    def issue(descs):
        for s, d, sem in descs:
            pltpu.async_copy(s, d, sem)

    def wait(descs):
        for s, d, sem in descs:
            pltpu.make_async_copy(s, d, sem).wait()

    issue(gather_descs(0))

    def chunk(k, _):
        @pl.when(k + 1 < NCH)
        def _():
            @pl.when(k >= 1)
            def _():
                wait(write_descs(k - 1))
            issue(gather_descs(k + 1))

        wait(gather_descs(k))
        off = (k % 2) * C

        def row(r, _):
            q = off + r
            for c in range(D // 16):
                s = c * 16
                gi[q, pl.ds(s, 16)] = gi[q, pl.ds(s, 16)] * gj[q, pl.ds(s, 16)]
            for c in range(HP // 16):
                s = c * 16
                gm[q, pl.ds(s, 16)] = gm[q, pl.ds(s, 16)] + ga[q, pl.ds(s, 16)]
            return 0

        lax.fori_loop(0, C, row, 0)
        issue(write_descs(k))
        return 0

    lax.fori_loop(0, NCH, chunk, 0)
    wait(write_descs(NCH - 2))
    wait(write_descs(NCH - 1))


def _mlp_kernel(x_ref, aff_ref, mid_ref, did_ref, gid_ref, sid_ref,
                w1c_ref, phi_ref, w2_ref, b2_ref, w3_ref, b3_ref,
                e_ref, den_ref):
    i = pl.program_id(0)
    x = x_ref[...]
    aff = aff_ref[...]
    col = lax.broadcasted_iota(jnp.int32, (1, HP), 1)
    not150 = (col != HID).astype(F32)
    is150 = (col == HID).astype(F32)

    d = did_ref[0, 0, :].reshape(B, 1)
    gd = gid_ref[0, 0, :].reshape(B, 1)
    sp = sid_ref[0, 0, :].reshape(B, 1)
    i48 = lax.broadcasted_iota(jnp.int32, (B, 48), 1)
    oh = ((i48 == d) | (i48 == gd + 16) | (i48 == sp + 32)).astype(F32)

    h1 = jnp.dot(x, w1c_ref[...], preferred_element_type=F32)
    h1 = h1 + jnp.dot(oh, phi_ref[...], preferred_element_type=F32)
    h1 = jnp.maximum(h1 + aff * not150, 0.0)
    h2 = jnp.maximum(jnp.dot(h1, w2_ref[...], preferred_element_type=F32)
                     + b2_ref[...], 0.0)
    sij = jnp.sum(h2 * w3_ref[...], axis=1, keepdims=True)
    sv = jnp.sum(aff * is150, axis=1, keepdims=True)
    coref = sij + b3_ref[...] + sv
    e = jnp.exp(coref)
    e_ref[...] = e.reshape(1, 1, B)

    mid = mid_ref[0, 0, :].reshape(B, 1)
    iN = lax.broadcasted_iota(jnp.int32, (B, N), 1)
    mask = (iN == mid).astype(F32)

    @pl.when(i == 0)
    def _():
        den_ref[...] = jnp.ones_like(den_ref)

    den_ref[...] += jnp.dot(e.reshape(1, B), mask, preferred_element_type=F32)


def _probs_kernel(e_ref, mid_ref, den_ref, p_ref, eps_ref):
    i = pl.program_id(0)
    den = den_ref[...]
    e = e_ref[0, 0, :].reshape(B, 1)
    mid = mid_ref[0, 0, :].reshape(B, 1)
    iN = lax.broadcasted_iota(jnp.int32, (B, N), 1)
    mask = (iN == mid).astype(F32)
    dsel = jnp.sum(mask * den, axis=1, keepdims=True)
    p_ref[...] = (e / dsel).reshape(1, 1, B)

    @pl.when(i == 0)
    def _():
        eps_ref[...] = 1.0 / den


def kernel(g_i, mention_scores, mention_ids, antecedent_ids, distance_ids,
           genre_ids, speaker_ids, W_dist, W_genre, W_speaker,
           W1, b1, W2, b2, W3, b3):
    pad = HP - HID
    w1aT = jnp.pad(W1[:, :D].T, ((0, 0), (0, pad)))
    w1bT = jnp.pad(W1[:, D:2 * D].T, ((0, 0), (0, pad)))
    w1cT = jnp.pad(W1[:, 2 * D:3 * D].T, ((0, 0), (0, pad)))
    w1dT = jnp.pad(W1[:, 3 * D:].T, ((0, 0), (0, pad)))
    b1p = jnp.pad(b1, (0, pad)).reshape(1, HP)
    b2p = jnp.pad(b2, (0, pad)).reshape(1, HP)
    w3row = jnp.pad(W3[0], (0, pad)).reshape(1, HP)
    b3a = b3.reshape(1, 1)
    e48 = jnp.zeros((48, 60), F32)
    e48 = e48.at[0:9, 0:20].set(W_dist)
    e48 = e48.at[16:24, 20:40].set(W_genre)
    e48 = e48.at[32:35, 40:60].set(W_speaker)

    gm, ga, phiT = pl.pallas_call(
        _prep_kernel,
        out_shape=[
            jax.ShapeDtypeStruct((N, HP), F32),
            jax.ShapeDtypeStruct((N, HP), F32),
            jax.ShapeDtypeStruct((48, HP), F32),
        ],
    )(g_i, w1aT, w1bT, mention_scores, b1p, e48, w1dT)

    sc = pl.kernel(
        _sc_body,
        out_type=[
            jax.ShapeDtypeStruct((P, D), F32),
            jax.ShapeDtypeStruct((P, HP), F32),
        ],
        mesh=plsc.VectorSubcoreMesh(core_axis_name="c", subcore_axis_name="s",
                                    num_cores=NC, num_subcores=NS),
        scratch_types=[
            pltpu.VMEM((PPW,), jnp.int32),
            pltpu.VMEM((PPW,), jnp.int32),
            pltpu.VMEM((2 * C, D), F32),
            pltpu.VMEM((2 * C, D), F32),
            pltpu.VMEM((2 * C, HP), F32),
            pltpu.VMEM((2 * C, HP), F32),
            pltpu.SemaphoreType.DMA,
            pltpu.SemaphoreType.DMA,
            pltpu.SemaphoreType.DMA,
            pltpu.SemaphoreType.DMA,
            pltpu.SemaphoreType.DMA,
            pltpu.SemaphoreType.DMA,
        ],
    )
    x, aff = sc(g_i, gm, ga, mention_ids, antecedent_ids)

    mid3 = mention_ids.reshape(NBLK, 1, B)
    did3 = distance_ids.reshape(NBLK, 1, B)
    gid3 = genre_ids.reshape(NBLK, 1, B)
    sid3 = speaker_ids.reshape(NBLK, 1, B)

    blk = lambda i: (i, 0, 0)
    full2 = lambda i: (0, 0)

    e3, den = pl.pallas_call(
        _mlp_kernel,
        grid=(NBLK,),
        in_specs=[
            pl.BlockSpec((B, D), lambda i: (i, 0)),
            pl.BlockSpec((B, HP), lambda i: (i, 0)),
            pl.BlockSpec((1, 1, B), blk),
            pl.BlockSpec((1, 1, B), blk),
            pl.BlockSpec((1, 1, B), blk),
            pl.BlockSpec((1, 1, B), blk),
            pl.BlockSpec((D, HP), full2),
            pl.BlockSpec((48, HP), full2),
            pl.BlockSpec((HP, HP), full2),
            pl.BlockSpec((1, HP), full2),
            pl.BlockSpec((1, HP), full2),
            pl.BlockSpec((1, 1), full2),
        ],
        out_specs=[
            pl.BlockSpec((1, 1, B), blk),
            pl.BlockSpec((1, N), full2),
        ],
        out_shape=[
            jax.ShapeDtypeStruct((NBLK, 1, B), F32),
            jax.ShapeDtypeStruct((1, N), F32),
        ],
    )(x, aff, mid3, did3, gid3, sid3, w1cT, phiT,
      jnp.pad(W2.T, ((0, pad), (0, pad))), b2p, w3row, b3a)

    p3, eps = pl.pallas_call(
        _probs_kernel,
        grid=(NBLK,),
        in_specs=[
            pl.BlockSpec((1, 1, B), blk),
            pl.BlockSpec((1, 1, B), blk),
            pl.BlockSpec((1, N), full2),
        ],
        out_specs=[
            pl.BlockSpec((1, 1, B), blk),
            pl.BlockSpec((1, N), full2),
        ],
        out_shape=[
            jax.ShapeDtypeStruct((NBLK, 1, B), F32),
            jax.ShapeDtypeStruct((1, N), F32),
        ],
    )(e3, mid3, den)

    return jnp.concatenate([p3.reshape(P), eps.reshape(N)])


# speedup vs baseline: 5.2329x; 1.0757x over previous
import jax
import jax.numpy as jnp
from jax import lax
from jax.experimental import pallas as pl
from jax.experimental.pallas import tpu as pltpu
from jax.experimental.pallas import tpu_sc as plsc

N = 2048
P = 16384
D = 1024
HID = 150
HP = 256
B = 256
NBLK = P // B
NC, NS = 2, 16
NW = NC * NS
H = P // 2
NBLKH = H // B
PPW = H // NW
C = 16
NCH = PPW // C
F32 = jnp.float32


def _prep_kernel(g_ref, w1a_ref, w1b_ref, ms_ref, b1_ref, e48_ref, w1d_ref,
                 gm_ref, ga_ref, phi_ref):
    g = g_ref[...]
    col = lax.broadcasted_iota(jnp.int32, (1, HP), 1)
    is150 = (col == HID).astype(F32)
    ms = ms_ref[...]
    gm_ref[...] = (jnp.dot(g, w1a_ref[...], preferred_element_type=F32)
                   + b1_ref[...] + ms * is150)
    ga_ref[...] = (jnp.dot(g, w1b_ref[...], preferred_element_type=F32)
                   + ms * is150)
    phi_ref[...] = jnp.dot(e48_ref[...], w1d_ref[...], preferred_element_type=F32)


def _sc_body(g_hbm, gm_hbm, ga_hbm, mid_hbm, aid_hbm, x_hbm, aff_hbm,
             midx, aidx, gi, gj, gm, ga, s_gi, s_gj, s_gm, s_ga, s_wx, s_wa):
    wid = lax.axis_index("s") * NC + lax.axis_index("c")
    base = wid * PPW
    pltpu.sync_copy(mid_hbm.at[pl.ds(base, PPW)], midx)
    pltpu.sync_copy(aid_hbm.at[pl.ds(base, PPW)], aidx)

    def gather_descs(k):
        off = (k % 2) * C
        i_idx = midx.at[pl.ds(k * C, C)]
        j_idx = aidx.at[pl.ds(k * C, C)]
        return (
            (g_hbm.at[i_idx], gi.at[pl.ds(off, C)], s_gi),
            (g_hbm.at[j_idx], gj.at[pl.ds(off, C)], s_gj),
            (gm_hbm.at[i_idx], gm.at[pl.ds(off, C)], s_gm),
            (ga_hbm.at[j_idx], ga.at[pl.ds(off, C)], s_ga),
        )

    def write_descs(k):
        off = (k % 2) * C
        row = base + k * C
        return (
            (gi.at[pl.ds(off, C)], x_hbm.at[pl.ds(row, C)], s_wx),
            (gm.at[pl.ds(off, C)], aff_hbm.at[pl.ds(row, C)], s_wa),
        )

    def issue(descs):
        for s, d, sem in descs:
            pltpu.async_copy(s, d, sem)

    def wait(descs):
        for s, d, sem in descs:
            pltpu.make_async_copy(s, d, sem).wait()

    issue(gather_descs(0))

    def chunk(k, _):
        @pl.when(k + 1 < NCH)
        def _():
            @pl.when(k >= 1)
            def _():
                wait(write_descs(k - 1))
            issue(gather_descs(k + 1))

        wait(gather_descs(k))
        off = (k % 2) * C

        def row(r, _):
            q = off + r
            for c in range(D // 16):
                s = c * 16
                gi[q, pl.ds(s, 16)] = gi[q, pl.ds(s, 16)] * gj[q, pl.ds(s, 16)]
            for c in range(HP // 16):
                s = c * 16
                gm[q, pl.ds(s, 16)] = gm[q, pl.ds(s, 16)] + ga[q, pl.ds(s, 16)]
            return 0

        lax.fori_loop(0, C, row, 0)
        issue(write_descs(k))
        return 0

    lax.fori_loop(0, NCH, chunk, 0)
    wait(write_descs(NCH - 2))
    wait(write_descs(NCH - 1))


def _mlp_kernel(x_ref, aff_ref, mid_ref, did_ref, gid_ref, sid_ref,
                w1c_ref, phi_ref, w2_ref, b2_ref, w3_ref, b3_ref,
                e_ref, den_ref):
    i = pl.program_id(0)
    x = x_ref[...]
    aff = aff_ref[...]
    col = lax.broadcasted_iota(jnp.int32, (1, HP), 1)
    not150 = (col != HID).astype(F32)
    is150 = (col == HID).astype(F32)

    d = did_ref[0, 0, :].reshape(B, 1)
    gd = gid_ref[0, 0, :].reshape(B, 1)
    sp = sid_ref[0, 0, :].reshape(B, 1)
    i48 = lax.broadcasted_iota(jnp.int32, (B, 48), 1)
    oh = ((i48 == d) | (i48 == gd + 16) | (i48 == sp + 32)).astype(F32)

    h1 = jnp.dot(x, w1c_ref[...], preferred_element_type=F32)
    h1 = h1 + jnp.dot(oh, phi_ref[...], preferred_element_type=F32)
    h1 = jnp.maximum(h1 + aff * not150, 0.0)
    h2 = jnp.maximum(jnp.dot(h1, w2_ref[...], preferred_element_type=F32)
                     + b2_ref[...], 0.0)
    sij = jnp.sum(h2 * w3_ref[...], axis=1, keepdims=True)
    sv = jnp.sum(aff * is150, axis=1, keepdims=True)
    coref = sij + b3_ref[...] + sv
    e = jnp.exp(coref)
    e_ref[...] = e.reshape(1, 1, B)

    mid = mid_ref[0, 0, :].reshape(B, 1)
    iN = lax.broadcasted_iota(jnp.int32, (B, N), 1)
    mask = (iN == mid).astype(F32)

    @pl.when(i == 0)
    def _():
        den_ref[...] = jnp.ones_like(den_ref)

    den_ref[...] += jnp.dot(e.reshape(1, B), mask, preferred_element_type=F32)


def _probs_kernel(e_ref, mid_ref, den0_ref, den1_ref, p_ref, eps_ref):
    i = pl.program_id(0)
    den = den0_ref[...] + den1_ref[...] - 1.0
    e = e_ref[0, 0, :].reshape(B, 1)
    mid = mid_ref[0, 0, :].reshape(B, 1)
    iN = lax.broadcasted_iota(jnp.int32, (B, N), 1)
    mask = (iN == mid).astype(F32)
    dsel = jnp.sum(mask * den, axis=1, keepdims=True)
    p_ref[...] = (e / dsel).reshape(1, 1, B)

    @pl.when(i == 0)
    def _():
        eps_ref[...] = 1.0 / den


def kernel(g_i, mention_scores, mention_ids, antecedent_ids, distance_ids,
           genre_ids, speaker_ids, W_dist, W_genre, W_speaker,
           W1, b1, W2, b2, W3, b3):
    pad = HP - HID
    w1aT = jnp.pad(W1[:, :D].T, ((0, 0), (0, pad)))
    w1bT = jnp.pad(W1[:, D:2 * D].T, ((0, 0), (0, pad)))
    w1cT = jnp.pad(W1[:, 2 * D:3 * D].T, ((0, 0), (0, pad)))
    w1dT = jnp.pad(W1[:, 3 * D:].T, ((0, 0), (0, pad)))
    b1p = jnp.pad(b1, (0, pad)).reshape(1, HP)
    b2p = jnp.pad(b2, (0, pad)).reshape(1, HP)
    w3row = jnp.pad(W3[0], (0, pad)).reshape(1, HP)
    b3a = b3.reshape(1, 1)
    e48 = jnp.zeros((48, 60), F32)
    e48 = e48.at[0:9, 0:20].set(W_dist)
    e48 = e48.at[16:24, 20:40].set(W_genre)
    e48 = e48.at[32:35, 40:60].set(W_speaker)

    gm, ga, phiT = pl.pallas_call(
        _prep_kernel,
        out_shape=[
            jax.ShapeDtypeStruct((N, HP), F32),
            jax.ShapeDtypeStruct((N, HP), F32),
            jax.ShapeDtypeStruct((48, HP), F32),
        ],
    )(g_i, w1aT, w1bT, mention_scores, b1p, e48, w1dT)

    sc = pl.kernel(
        _sc_body,
        out_type=[
            jax.ShapeDtypeStruct((H, D), F32),
            jax.ShapeDtypeStruct((H, HP), F32),
        ],
        mesh=plsc.VectorSubcoreMesh(core_axis_name="c", subcore_axis_name="s",
                                    num_cores=NC, num_subcores=NS),
        scratch_types=[
            pltpu.VMEM((PPW,), jnp.int32),
            pltpu.VMEM((PPW,), jnp.int32),
            pltpu.VMEM((2 * C, D), F32),
            pltpu.VMEM((2 * C, D), F32),
            pltpu.VMEM((2 * C, HP), F32),
            pltpu.VMEM((2 * C, HP), F32),
            pltpu.SemaphoreType.DMA,
            pltpu.SemaphoreType.DMA,
            pltpu.SemaphoreType.DMA,
            pltpu.SemaphoreType.DMA,
            pltpu.SemaphoreType.DMA,
            pltpu.SemaphoreType.DMA,
        ],
    )
    mid3 = mention_ids.reshape(NBLK, 1, B)
    did3 = distance_ids.reshape(NBLK, 1, B)
    gid3 = genre_ids.reshape(NBLK, 1, B)
    sid3 = speaker_ids.reshape(NBLK, 1, B)
    w2T = jnp.pad(W2.T, ((0, pad), (0, pad)))

    blk = lambda i: (i, 0, 0)
    full2 = lambda i: (0, 0)

    mlp = pl.pallas_call(
        _mlp_kernel,
        grid=(NBLKH,),
        in_specs=[
            pl.BlockSpec((B, D), lambda i: (i, 0)),
            pl.BlockSpec((B, HP), lambda i: (i, 0)),
            pl.BlockSpec((1, 1, B), blk),
            pl.BlockSpec((1, 1, B), blk),
            pl.BlockSpec((1, 1, B), blk),
            pl.BlockSpec((1, 1, B), blk),
            pl.BlockSpec((D, HP), full2),
            pl.BlockSpec((48, HP), full2),
            pl.BlockSpec((HP, HP), full2),
            pl.BlockSpec((1, HP), full2),
            pl.BlockSpec((1, HP), full2),
            pl.BlockSpec((1, 1), full2),
        ],
        out_specs=[
            pl.BlockSpec((1, 1, B), blk),
            pl.BlockSpec((1, N), full2),
        ],
        out_shape=[
            jax.ShapeDtypeStruct((NBLKH, 1, B), F32),
            jax.ShapeDtypeStruct((1, N), F32),
        ],
    )

    es, dens = [], []
    for h in range(2):
        lo = h * H
        x, aff = sc(g_i, gm, ga,
                    lax.slice(mention_ids, (lo,), (lo + H,)),
                    lax.slice(antecedent_ids, (lo,), (lo + H,)))
        hb = h * NBLKH
        e3, den = mlp(x, aff,
                      mid3[hb:hb + NBLKH], did3[hb:hb + NBLKH],
                      gid3[hb:hb + NBLKH], sid3[hb:hb + NBLKH],
                      w1cT, phiT, w2T, b2p, w3row, b3a)
        es.append(e3)
        dens.append(den)

    e3 = jnp.concatenate(es, axis=0)

    p3, eps = pl.pallas_call(
        _probs_kernel,
        grid=(NBLK,),
        in_specs=[
            pl.BlockSpec((1, 1, B), blk),
            pl.BlockSpec((1, 1, B), blk),
            pl.BlockSpec((1, N), full2),
            pl.BlockSpec((1, N), full2),
        ],
        out_specs=[
            pl.BlockSpec((1, 1, B), blk),
            pl.BlockSpec((1, N), full2),
        ],
        out_shape=[
            jax.ShapeDtypeStruct((NBLK, 1, B), F32),
            jax.ShapeDtypeStruct((1, N), F32),
        ],
    )(e3, mid3, dens[0], dens[1])

    return jnp.concatenate([p3.reshape(P), eps.reshape(N)])
